# baseline (speedup 1.0000x reference)
import functools

import jax
import jax.numpy as jnp
from jax import lax
from jax.experimental import pallas as pl
from jax.experimental.pallas import tpu as pltpu
from jax.experimental.pallas import tpu_sc as plsc

NC = 2
NS = 16
NW = NC * NS
NSEQ = 8


@functools.partial(jax.jit, static_argnames=("d",))
def _gather(table, xi, *, d):
    n_seq, seq_len = xi.shape
    assert seq_len <= 128
    s_per_w = n_seq // NW
    n_groups = s_per_w // NSEQ
    assert n_groups % 2 == 0
    mesh = plsc.VectorSubcoreMesh(core_axis_name="c", subcore_axis_name="s")

    @functools.partial(
        pl.kernel,
        mesh=mesh,
        compiler_params=pltpu.CompilerParams(use_tc_tiling_on_sc=False),
        out_type=jax.ShapeDtypeStruct((n_seq, seq_len, d), jnp.float32),
        scratch_types=[
            pltpu.VMEM((s_per_w, seq_len), jnp.int32),
            pltpu.VMEM((2, NSEQ, seq_len, d), jnp.float32),
            [pltpu.SemaphoreType.DMA] * 2,
            [pltpu.SemaphoreType.DMA] * 2,
        ],
    )
    def k(table_hbm, idx_hbm, out_hbm, idx_v, rows_v, gsems, wsems):
        wid = lax.axis_index("s") * NC + lax.axis_index("c")
        base = wid * s_per_w
        pltpu.sync_copy(idx_hbm.at[pl.ds(base, s_per_w)], idx_v)

        def pair(g2, carry):
            for p in range(2):
                g = 2 * g2 + p

                @pl.when(g2 > 0)
                def _wait_write(p=p):
                    pltpu.make_async_copy(
                        rows_v.at[p], out_hbm.at[pl.ds(0, NSEQ)], wsems[p]
                    ).wait()

                for q in range(NSEQ):
                    pltpu.async_copy(
                        table_hbm.at[idx_v.at[g * NSEQ + q]],
                        rows_v.at[p, q],
                        gsems[p],
                    )
            for p in range(2):
                g = 2 * g2 + p
                pltpu.make_async_copy(
                    out_hbm.at[pl.ds(0, NSEQ)], rows_v.at[p], gsems[p]
                ).wait()
                pltpu.async_copy(
                    rows_v.at[p],
                    out_hbm.at[pl.ds(base + g * NSEQ, NSEQ)],
                    wsems[p],
                )
            return carry

        lax.fori_loop(0, n_groups // 2, pair, 0)
        for p in range(2):
            pltpu.make_async_copy(
                rows_v.at[p], out_hbm.at[pl.ds(0, NSEQ)], wsems[p]
            ).wait()

    return k(table, xi)


K_SLICES = 4


def kernel(x, table):
    n_seq, seq_len = x.shape
    d = table.shape[1]
    xi = x.astype(jnp.int32)
    per = NW * 2 * NSEQ
    k = K_SLICES if n_seq % (K_SLICES * per) == 0 else 1
    pad = (-n_seq) % per
    if pad:
        xi = jnp.concatenate([xi, jnp.zeros((pad, seq_len), jnp.int32)])
    sl = xi.shape[0] // k
    outs = [_gather(table, xi[i * sl:(i + 1) * sl], d=d) for i in range(k)]
    out = outs[0] if k == 1 else jnp.concatenate(outs, axis=0)
    if pad:
        out = out[:n_seq]
    return out


# speedup vs baseline: 1.0500x; 1.0500x over previous
import functools

import jax
import jax.numpy as jnp
from jax import lax
from jax.experimental import pallas as pl
from jax.experimental.pallas import tpu as pltpu
from jax.experimental.pallas import tpu_sc as plsc

NC = 2
NS = 16
NW = NC * NS
CHUNK = 128
NBUF = 4


@functools.partial(jax.jit, static_argnames=("n_chunks", "d"))
def _gather(table, idx, *, n_chunks, d):
    assert n_chunks % NBUF == 0
    n_groups = n_chunks // NBUF
    b_per_w = n_chunks * CHUNK
    total = NW * b_per_w
    mesh = plsc.VectorSubcoreMesh(core_axis_name="c", subcore_axis_name="s")

    @functools.partial(
        pl.kernel,
        mesh=mesh,
        compiler_params=pltpu.CompilerParams(use_tc_tiling_on_sc=False),
        out_type=jax.ShapeDtypeStruct((total, d), jnp.float32),
        scratch_types=[
            pltpu.VMEM((n_chunks, CHUNK), jnp.int32),
            pltpu.VMEM((NBUF, CHUNK, d), jnp.float32),
            [pltpu.SemaphoreType.DMA] * NBUF,
            [pltpu.SemaphoreType.DMA] * NBUF,
        ],
    )
    def k(table_hbm, idx_hbm, out_hbm, idx_v, rows_v, gsems, wsems):
        wid = lax.axis_index("s") * NC + lax.axis_index("c")
        base = wid * b_per_w
        pltpu.sync_copy(idx_hbm.at[wid], idx_v)

        def group(g, carry):
            for b in range(NBUF):
                j = g * NBUF + b

                @pl.when(g > 0)
                def _wait_write(b=b):
                    pltpu.make_async_copy(
                        rows_v.at[b], out_hbm.at[pl.ds(0, CHUNK)], wsems[b]
                    ).wait()

                pltpu.async_copy(table_hbm.at[idx_v.at[j]], rows_v.at[b], gsems[b])
            for b in range(NBUF):
                j = g * NBUF + b
                pltpu.make_async_copy(
                    table_hbm.at[pl.ds(0, CHUNK)], rows_v.at[b], gsems[b]
                ).wait()
                pltpu.async_copy(
                    rows_v.at[b], out_hbm.at[pl.ds(base + j * CHUNK, CHUNK)], wsems[b]
                )
            return carry

        lax.fori_loop(0, n_groups, group, 0)
        for b in range(NBUF):
            pltpu.make_async_copy(
                rows_v.at[b], out_hbm.at[pl.ds(0, CHUNK)], wsems[b]
            ).wait()

    return k(table, idx)


def kernel(x, table):
    d = table.shape[1]
    flat = x.reshape(-1).astype(jnp.int32)
    b = flat.shape[0]
    per = NW * CHUNK * NBUF
    pad = (-b) % per
    if pad:
        flat = jnp.concatenate([flat, jnp.zeros((pad,), jnp.int32)])
    n_chunks = flat.shape[0] // (NW * CHUNK)
    idx = flat.reshape(NW, n_chunks, CHUNK)
    out = _gather(table, idx, n_chunks=n_chunks, d=d)
    if pad:
        out = out[:b]
    return out.reshape(*x.shape, d)


# speedup vs baseline: 1.0557x; 1.0054x over previous
import functools

import jax
import jax.numpy as jnp
from jax import lax
from jax.experimental import pallas as pl
from jax.experimental.pallas import tpu as pltpu
from jax.experimental.pallas import tpu_sc as plsc

NC = 2
NS = 16
NW = NC * NS
CHUNK = 128
NBUF = 8


@functools.partial(jax.jit, static_argnames=("n_chunks", "d"))
def _gather(table, idx, *, n_chunks, d):
    assert n_chunks % NBUF == 0
    n_groups = n_chunks // NBUF
    b_per_w = n_chunks * CHUNK
    total = NW * b_per_w
    mesh = plsc.VectorSubcoreMesh(core_axis_name="c", subcore_axis_name="s")

    @functools.partial(
        pl.kernel,
        mesh=mesh,
        compiler_params=pltpu.CompilerParams(use_tc_tiling_on_sc=False),
        out_type=jax.ShapeDtypeStruct((total, d), jnp.float32),
        scratch_types=[
            pltpu.VMEM((n_chunks, CHUNK), jnp.int32),
            pltpu.VMEM((NBUF, CHUNK, d), jnp.float32),
            [pltpu.SemaphoreType.DMA] * NBUF,
            [pltpu.SemaphoreType.DMA] * NBUF,
        ],
    )
    def k(table_hbm, idx_hbm, out_hbm, idx_v, rows_v, gsems, wsems):
        wid = lax.axis_index("s") * NC + lax.axis_index("c")
        base = wid * b_per_w
        pltpu.sync_copy(idx_hbm.at[wid], idx_v)

        def group(g, carry):
            for b in range(NBUF):
                j = g * NBUF + b

                @pl.when(g > 0)
                def _wait_write(b=b):
                    pltpu.make_async_copy(
                        rows_v.at[b], out_hbm.at[pl.ds(0, CHUNK)], wsems[b]
                    ).wait()

                pltpu.async_copy(table_hbm.at[idx_v.at[j]], rows_v.at[b], gsems[b])
            for b in range(NBUF):
                j = g * NBUF + b
                pltpu.make_async_copy(
                    table_hbm.at[pl.ds(0, CHUNK)], rows_v.at[b], gsems[b]
                ).wait()
                pltpu.async_copy(
                    rows_v.at[b], out_hbm.at[pl.ds(base + j * CHUNK, CHUNK)], wsems[b]
                )
            return carry

        lax.fori_loop(0, n_groups, group, 0)
        for b in range(NBUF):
            pltpu.make_async_copy(
                rows_v.at[b], out_hbm.at[pl.ds(0, CHUNK)], wsems[b]
            ).wait()

    return k(table, idx)


def kernel(x, table):
    d = table.shape[1]
    flat = x.reshape(-1).astype(jnp.int32)
    b = flat.shape[0]
    per = NW * CHUNK * NBUF
    pad = (-b) % per
    if pad:
        flat = jnp.concatenate([flat, jnp.zeros((pad,), jnp.int32)])
    n_chunks = flat.shape[0] // (NW * CHUNK)
    idx = flat.reshape(NW, n_chunks, CHUNK)
    out = _gather(table, idx, n_chunks=n_chunks, d=d)
    if pad:
        out = out[:b]
    return out.reshape(*x.shape, d)


# speedup vs baseline: 1.0576x; 1.0017x over previous
import functools

import jax
import jax.numpy as jnp
from jax import lax
from jax.experimental import pallas as pl
from jax.experimental.pallas import tpu as pltpu
from jax.experimental.pallas import tpu_sc as plsc

NC = 2
NS = 16
NW = NC * NS
CHUNK = 128
NBUF = 8


@functools.partial(jax.jit, static_argnames=("d",))
def _gather(table, xt, *, d):
    seq_len, n_seq = xt.shape
    s_per_w = n_seq // NW
    sblocks = s_per_w // CHUNK
    n_chunks = seq_len * sblocks
    assert n_chunks % NBUF == 0
    n_groups = n_chunks // NBUF
    mesh = plsc.VectorSubcoreMesh(core_axis_name="c", subcore_axis_name="s")

    @functools.partial(
        pl.kernel,
        mesh=mesh,
        compiler_params=pltpu.CompilerParams(use_tc_tiling_on_sc=False),
        out_type=jax.ShapeDtypeStruct((n_seq, seq_len, d), jnp.float32),
        scratch_types=[
            pltpu.VMEM((seq_len, s_per_w), jnp.int32),
            pltpu.VMEM((NBUF, CHUNK, d), jnp.float32),
            [pltpu.SemaphoreType.DMA] * NBUF,
            [pltpu.SemaphoreType.DMA] * NBUF,
        ],
    )
    def k(table_hbm, xt_hbm, out_hbm, idx_v, rows_v, gsems, wsems):
        wid = lax.axis_index("s") * NC + lax.axis_index("c")
        base = wid * s_per_w
        pltpu.sync_copy(xt_hbm.at[:, pl.ds(base, s_per_w)], idx_v)

        def group(g, carry):
            for b in range(NBUF):
                j = g * NBUF + b
                p = j // sblocks
                sb = j % sblocks

                @pl.when(g > 0)
                def _wait_write(b=b):
                    pltpu.make_async_copy(
                        rows_v.at[b], out_hbm.at[pl.ds(0, CHUNK), 0], wsems[b]
                    ).wait()

                pltpu.async_copy(
                    table_hbm.at[idx_v.at[p, pl.ds(sb * CHUNK, CHUNK)]],
                    rows_v.at[b],
                    gsems[b],
                )
            for b in range(NBUF):
                j = g * NBUF + b
                p = j // sblocks
                sb = j % sblocks
                pltpu.make_async_copy(
                    table_hbm.at[pl.ds(0, CHUNK)], rows_v.at[b], gsems[b]
                ).wait()
                pltpu.async_copy(
                    rows_v.at[b],
                    out_hbm.at[pl.ds(base + sb * CHUNK, CHUNK), p],
                    wsems[b],
                )
            return carry

        lax.fori_loop(0, n_groups, group, 0)
        for b in range(NBUF):
            pltpu.make_async_copy(
                rows_v.at[b], out_hbm.at[pl.ds(0, CHUNK), 0], wsems[b]
            ).wait()

    return k(table, xt)


def kernel(x, table):
    n_seq, seq_len = x.shape
    d = table.shape[1]
    xt = jnp.transpose(x.astype(jnp.int32))
    per = NW * CHUNK
    pad = (-n_seq) % per
    if pad:
        xt = jnp.concatenate([xt, jnp.zeros((seq_len, pad), jnp.int32)], axis=1)
    if (seq_len * ((n_seq + pad) // per)) % NBUF:
        extra = 0
        while (seq_len * ((n_seq + pad + extra) // per)) % NBUF:
            extra += per
        xt = jnp.concatenate([xt, jnp.zeros((seq_len, extra), jnp.int32)], axis=1)
        pad += extra
    out = _gather(table, xt, d=d)
    if pad:
        out = out[:n_seq]
    return out
